# baseline (speedup 1.0000x reference)
import functools

import jax
import jax.numpy as jnp
from jax import lax
from jax.experimental import pallas as pl
from jax.experimental.pallas import tpu as pltpu
from jax.experimental.pallas import tpu_sc as plsc

S = 2048
D = 768
H = 12
DH = 64
F = 3072
E = 8
BM = 1024
BLK = 768
NQ = S // BM
NBLK = (2 * S + E * (BLK - 1) + BLK - 1) // BLK
NPAD = NBLK * BLK
EPS = 1e-5
NEG = -1e9
GP = 128


def _ln(xb, w, b):
    mu = jnp.mean(xb, axis=-1, keepdims=True)
    xc = xb - mu
    var = jnp.mean(xc * xc, axis=-1, keepdims=True)
    return xc * lax.rsqrt(var + EPS) * w + b


def _qkv_kernel(x_ref, lw_ref, lb_ref, wqkv_ref, bqkv_ref, qkv_ref):
    xn = _ln(x_ref[...], lw_ref[...], lb_ref[...])
    qkv_ref[...] = (
        jnp.dot(xn, wqkv_ref[...], preferred_element_type=jnp.float32)
        + bqkv_ref[...]
    )


BMA = 512
NQA = S // BMA


def _attn_kernel(q_ref, k_ref, v_ref, ctx_ref):
    qi = pl.program_id(1)
    q = q_ref[0]
    scale = 1.0 / (DH ** 0.5)

    def make_branch(nb):
        w = nb * BMA

        def br():
            k = k_ref[0, :w, :]
            v = v_ref[0, :w, :]
            s = lax.dot_general(q, k, (((1,), (1,)), ((), ())),
                                preferred_element_type=jnp.float32) * scale
            rows = qi * BMA + lax.broadcasted_iota(jnp.int32, (BMA, w), 0)
            cols = lax.broadcasted_iota(jnp.int32, (BMA, w), 1)
            s = jnp.where(cols <= rows, s, NEG)
            m = jnp.max(s, axis=1, keepdims=True)
            p = jnp.exp(s - m)
            l = jnp.sum(p, axis=1, keepdims=True)
            return jnp.dot(p, v, preferred_element_type=jnp.float32) / l
        return br

    ctx_ref[0] = lax.switch(qi, [make_branch(i + 1) for i in range(NQA)])


def _post_kernel(ctx_ref, x_ref, wo_ref, bo_ref, lw_ref, lb_ref, wg_ref,
                 x1_ref, xn2_ref, idx_ref, gw_ref):
    x1 = (jnp.dot(ctx_ref[...], wo_ref[...],
                  preferred_element_type=jnp.float32)
          + bo_ref[...] + x_ref[...])
    x1_ref[...] = x1
    xn2 = _ln(x1, lw_ref[...], lb_ref[...])
    xn2_ref[...] = xn2
    g = jnp.dot(xn2, wg_ref[...], preferred_element_type=jnp.float32)
    colid = lax.broadcasted_iota(jnp.int32, (BM, GP), 1)
    g = jnp.where(colid < E, g, NEG)
    v1 = jnp.max(g, axis=1, keepdims=True)
    i1 = jnp.min(jnp.where(g == v1, colid, GP), axis=1, keepdims=True)
    gm = jnp.where(colid == i1, NEG, g)
    v2 = jnp.max(gm, axis=1, keepdims=True)
    i2 = jnp.min(jnp.where(gm == v2, colid, GP), axis=1, keepdims=True)
    e2 = jnp.exp(v2 - v1)
    w1 = 1.0 / (1.0 + e2)
    col8 = lax.broadcasted_iota(jnp.int32, (BM, 8), 1)
    idx_ref[...] = jnp.where(col8 == 0, i1, i2)
    gw_ref[...] = jnp.where(col8 == 0, w1, 1.0 - w1)


_NW = 32


def _make_sc_gather(nrows):
    rows_w = nrows // _NW
    nch = -(-rows_w // 40)
    crows = rows_w // nch
    mesh = plsc.VectorSubcoreMesh(core_axis_name="c", subcore_axis_name="s")

    @functools.partial(
        pl.kernel,
        out_type=jax.ShapeDtypeStruct((nrows, D), jnp.float32),
        mesh=mesh,
        scratch_types=[
            pltpu.VMEM((nch, crows), jnp.int32),
            pltpu.VMEM((nch, crows, D), jnp.float32),
            pltpu.SemaphoreType.DMA,
            pltpu.SemaphoreType.DMA,
        ],
    )
    def gather(src_hbm, ids_hbm, out_hbm, idx_v, rows_v, gsem, wsem):
        wid = lax.axis_index("s") * 2 + lax.axis_index("c")
        base = wid * rows_w
        for c in range(nch):
            pltpu.sync_copy(ids_hbm.at[pl.ds(base + c * crows, crows)],
                            idx_v.at[c])
        gcps = [pltpu.async_copy(src_hbm.at[idx_v.at[c]], rows_v.at[c], gsem)
                for c in range(nch)]
        wcps = []
        for c in range(nch):
            gcps[c].wait()
            wcps.append(pltpu.async_copy(
                rows_v.at[c], out_hbm.at[pl.ds(base + c * crows, crows)],
                wsem))
        for cp in wcps:
            cp.wait()

    return gather


_sc_gather_tok = _make_sc_gather(2 * S)

_PPW = 2 * S // _NW


def _sc_dispatch_kernel(src_hbm, slots_hbm, out_hbm, idx_v, rows_v, sem):
    wid = lax.axis_index("s") * 2 + lax.axis_index("c")
    pltpu.sync_copy(slots_hbm.at[wid], idx_v)
    tok0 = lax.rem(wid * _PPW, S)
    pltpu.sync_copy(src_hbm.at[pl.ds(tok0, _PPW)], rows_v)
    pltpu.async_copy(rows_v, out_hbm.at[idx_v], sem).wait()


_sc_dispatch = functools.partial(
    pl.kernel,
    out_type=jax.ShapeDtypeStruct((NPAD, D), jnp.float32),
    mesh=plsc.VectorSubcoreMesh(core_axis_name="c", subcore_axis_name="s"),
    scratch_types=[
        pltpu.VMEM((_PPW,), jnp.int32),
        pltpu.VMEM((_PPW, D), jnp.float32),
        pltpu.SemaphoreType.DMA,
    ],
)(_sc_dispatch_kernel)


def _ffn_kernel(be_ref, nv_ref, xs_ref, w1_ref, b1_ref, w2_ref, b2_ref,
                o_ref):
    b = pl.program_id(0)

    @pl.when(b < nv_ref[0])
    def _():
        xs = xs_ref[...]
        h = (jnp.dot(xs, w1_ref[0], preferred_element_type=jnp.float32)
             + b1_ref[0])
        h = 0.5 * h * (1.0 + lax.erf(h * (2.0 ** -0.5)))
        o_ref[...] = (
            jnp.dot(h, w2_ref[0], preferred_element_type=jnp.float32)
            + b2_ref[0])


def _add3_kernel(x1_ref, g0_ref, g1_ref, gw_ref, y_ref):
    w0 = gw_ref[:, 0:1]
    w1 = gw_ref[:, 1:2]
    y_ref[...] = x1_ref[...] + w0 * g0_ref[0] + w1 * g1_ref[0]


def _routing(i1, i2):
    experts = jnp.concatenate([i1, i2])
    onehot = (experts[:, None] == jnp.arange(E, dtype=jnp.int32)[None, :])
    onehot = onehot.astype(jnp.int32)
    rank = jnp.sum((jnp.cumsum(onehot, axis=0) - onehot) * onehot, axis=1)
    counts = jnp.sum(onehot, axis=0)
    pad_counts = ((counts + BLK - 1) // BLK) * BLK
    cum_pad = jnp.cumsum(pad_counts)
    pad_off = jnp.concatenate(
        [jnp.zeros(1, jnp.int32), cum_pad[:-1].astype(jnp.int32)])
    slot = pad_off[experts] + rank
    nvalid = (cum_pad[-1] // BLK).astype(jnp.int32)
    blk_starts = jnp.arange(NBLK, dtype=jnp.int32) * BLK
    block_expert = jnp.clip(
        jnp.searchsorted(cum_pad, blk_starts, side="right"), 0, E - 1
    ).astype(jnp.int32)
    return block_expert, nvalid, slot


def kernel(x, ln1_w, ln1_b, ln2_w, ln2_b, Wqkv, bqkv, Wo, bo, Wg, W1, b1,
           W2, b2):
    x2 = x.reshape(S, D)
    row2 = lambda a: a.reshape(1, -1)

    qkv = pl.pallas_call(
        _qkv_kernel,
        grid=(NQ,),
        in_specs=[
            pl.BlockSpec((BM, D), lambda i: (i, 0)),
            pl.BlockSpec((1, D), lambda i: (0, 0)),
            pl.BlockSpec((1, D), lambda i: (0, 0)),
            pl.BlockSpec((D, 3 * D), lambda i: (0, 0)),
            pl.BlockSpec((1, 3 * D), lambda i: (0, 0)),
        ],
        out_specs=pl.BlockSpec((BM, 3 * D), lambda i: (i, 0)),
        out_shape=jax.ShapeDtypeStruct((S, 3 * D), jnp.float32),
    )(x2, row2(ln1_w), row2(ln1_b), Wqkv, row2(bqkv))

    qkvr = qkv.reshape(S, 3, H, DH).transpose(1, 2, 0, 3)
    q3, k3, v3 = qkvr[0], qkvr[1], qkvr[2]
    ctxh = pl.pallas_call(
        _attn_kernel,
        grid=(H, NQA),
        in_specs=[
            pl.BlockSpec((1, BMA, DH), lambda h, qi: (h, qi, 0)),
            pl.BlockSpec((1, S, DH), lambda h, qi: (h, 0, 0)),
            pl.BlockSpec((1, S, DH), lambda h, qi: (h, 0, 0)),
        ],
        out_specs=pl.BlockSpec((1, BMA, DH), lambda h, qi: (h, qi, 0)),
        out_shape=jax.ShapeDtypeStruct((H, S, DH), jnp.float32),
    )(q3, k3, v3)
    ctx = ctxh.transpose(1, 0, 2).reshape(S, D)

    Wg_pad = jnp.zeros((D, GP), jnp.float32).at[:, :E].set(Wg)
    outs = pl.pallas_call(
        _post_kernel,
        grid=(NQ,),
        in_specs=[
            pl.BlockSpec((BM, D), lambda i: (i, 0)),
            pl.BlockSpec((BM, D), lambda i: (i, 0)),
            pl.BlockSpec((D, D), lambda i: (0, 0)),
            pl.BlockSpec((1, D), lambda i: (0, 0)),
            pl.BlockSpec((1, D), lambda i: (0, 0)),
            pl.BlockSpec((1, D), lambda i: (0, 0)),
            pl.BlockSpec((D, GP), lambda i: (0, 0)),
        ],
        out_specs=[
            pl.BlockSpec((BM, D), lambda i: (i, 0)),
            pl.BlockSpec((BM, D), lambda i: (i, 0)),
            pl.BlockSpec((BM, 8), lambda i: (i, 0)),
            pl.BlockSpec((BM, 8), lambda i: (i, 0)),
        ],
        out_shape=[
            jax.ShapeDtypeStruct((S, D), jnp.float32),
            jax.ShapeDtypeStruct((S, D), jnp.float32),
            jax.ShapeDtypeStruct((S, 8), jnp.int32),
            jax.ShapeDtypeStruct((S, 8), jnp.float32),
        ],
    )(ctx, x2, Wo, row2(bo), row2(ln2_w), row2(ln2_b), Wg_pad)
    x1, xn2, idxm, gwm = outs

    block_expert, nvalid, slot01 = _routing(idxm[:, 0], idxm[:, 1])
    nv = nvalid.reshape(1)

    xs = _sc_dispatch(xn2, slot01.reshape(_NW, _PPW))

    o = pl.pallas_call(
        _ffn_kernel,
        grid_spec=pltpu.PrefetchScalarGridSpec(
            num_scalar_prefetch=2,
            grid=(NBLK,),
            in_specs=[
                pl.BlockSpec((BLK, D), lambda b, be, nvr: (b, 0)),
                pl.BlockSpec((1, D, F), lambda b, be, nvr: (be[b], 0, 0)),
                pl.BlockSpec((1, 1, F), lambda b, be, nvr: (be[b], 0, 0)),
                pl.BlockSpec((1, F, D), lambda b, be, nvr: (be[b], 0, 0)),
                pl.BlockSpec((1, 1, D), lambda b, be, nvr: (be[b], 0, 0)),
            ],
            out_specs=pl.BlockSpec((BLK, D), lambda b, be, nvr: (b, 0)),
        ),
        out_shape=jax.ShapeDtypeStruct((NPAD, D), jnp.float32),
    )(block_expert, nv, xs, W1, b1.reshape(E, 1, F), W2,
      b2.reshape(E, 1, D))

    g01 = _sc_gather_tok(o, slot01).reshape(2, S, D)

    y = pl.pallas_call(
        _add3_kernel,
        grid=(NQ,),
        in_specs=[
            pl.BlockSpec((BM, D), lambda i: (i, 0)),
            pl.BlockSpec((1, BM, D), lambda i: (0, i, 0)),
            pl.BlockSpec((1, BM, D), lambda i: (1, i, 0)),
            pl.BlockSpec((BM, 8), lambda i: (i, 0)),
        ],
        out_specs=pl.BlockSpec((BM, D), lambda i: (i, 0)),
        out_shape=jax.ShapeDtypeStruct((S, D), jnp.float32),
    )(x1, g01, g01, gwm)

    return y.reshape(1, S, D)


# speedup vs baseline: 1.0016x; 1.0016x over previous
import functools

import jax
import jax.numpy as jnp
from jax import lax
from jax.experimental import pallas as pl
from jax.experimental.pallas import tpu as pltpu
from jax.experimental.pallas import tpu_sc as plsc

S = 2048
D = 768
H = 12
DH = 64
F = 3072
E = 8
BM = 512
BLK = 768
NQ = S // BM
NBLK = (2 * S + E * (BLK - 1) + BLK - 1) // BLK
NPAD = NBLK * BLK
EPS = 1e-5
NEG = -1e9
GP = 128


def _ln(xb, w, b):
    mu = jnp.mean(xb, axis=-1, keepdims=True)
    xc = xb - mu
    var = jnp.mean(xc * xc, axis=-1, keepdims=True)
    return xc * lax.rsqrt(var + EPS) * w + b


def _qkv_kernel(x_ref, lw_ref, lb_ref, wqkv_ref, bqkv_ref, qkv_ref):
    xn = _ln(x_ref[...], lw_ref[...], lb_ref[...])
    qkv_ref[...] = (
        jnp.dot(xn, wqkv_ref[...], preferred_element_type=jnp.float32)
        + bqkv_ref[...]
    )


BMA = 512
NQA = S // BMA


def _attn_kernel(q_ref, k_ref, v_ref, ctx_ref):
    qi = pl.program_id(1)
    q = q_ref[0]
    scale = 1.0 / (DH ** 0.5)

    def make_branch(nb):
        w = nb * BMA

        def br():
            k = k_ref[0, :w, :]
            v = v_ref[0, :w, :]
            s = lax.dot_general(q, k, (((1,), (1,)), ((), ())),
                                preferred_element_type=jnp.float32) * scale
            rows = qi * BMA + lax.broadcasted_iota(jnp.int32, (BMA, w), 0)
            cols = lax.broadcasted_iota(jnp.int32, (BMA, w), 1)
            s = jnp.where(cols <= rows, s, NEG)
            m = jnp.max(s, axis=1, keepdims=True)
            p = jnp.exp(s - m)
            l = jnp.sum(p, axis=1, keepdims=True)
            return jnp.dot(p, v, preferred_element_type=jnp.float32) / l
        return br

    ctx_ref[0] = lax.switch(qi, [make_branch(i + 1) for i in range(NQA)])


def _post_kernel(ctx_ref, x_ref, wo_ref, bo_ref, lw_ref, lb_ref, wg_ref,
                 x1_ref, xn2_ref, idx_ref, gw_ref):
    x1 = (jnp.dot(ctx_ref[...], wo_ref[...],
                  preferred_element_type=jnp.float32)
          + bo_ref[...] + x_ref[...])
    x1_ref[...] = x1
    xn2 = _ln(x1, lw_ref[...], lb_ref[...])
    xn2_ref[...] = xn2
    g = jnp.dot(xn2, wg_ref[...], preferred_element_type=jnp.float32)
    colid = lax.broadcasted_iota(jnp.int32, (BM, GP), 1)
    g = jnp.where(colid < E, g, NEG)
    v1 = jnp.max(g, axis=1, keepdims=True)
    i1 = jnp.min(jnp.where(g == v1, colid, GP), axis=1, keepdims=True)
    gm = jnp.where(colid == i1, NEG, g)
    v2 = jnp.max(gm, axis=1, keepdims=True)
    i2 = jnp.min(jnp.where(gm == v2, colid, GP), axis=1, keepdims=True)
    e2 = jnp.exp(v2 - v1)
    w1 = 1.0 / (1.0 + e2)
    col8 = lax.broadcasted_iota(jnp.int32, (BM, 8), 1)
    idx_ref[...] = jnp.where(col8 == 0, i1, i2)
    gw_ref[...] = jnp.where(col8 == 0, w1, 1.0 - w1)


_NW = 32


def _make_sc_gather(nrows):
    rows_w = nrows // _NW
    nch = -(-rows_w // 40)
    crows = rows_w // nch
    mesh = plsc.VectorSubcoreMesh(core_axis_name="c", subcore_axis_name="s")

    @functools.partial(
        pl.kernel,
        out_type=jax.ShapeDtypeStruct((nrows, D), jnp.float32),
        mesh=mesh,
        scratch_types=[
            pltpu.VMEM((nch, crows), jnp.int32),
            pltpu.VMEM((nch, crows, D), jnp.float32),
            pltpu.SemaphoreType.DMA,
            pltpu.SemaphoreType.DMA,
        ],
    )
    def gather(src_hbm, ids_hbm, out_hbm, idx_v, rows_v, gsem, wsem):
        wid = lax.axis_index("s") * 2 + lax.axis_index("c")
        base = wid * rows_w
        for c in range(nch):
            pltpu.sync_copy(ids_hbm.at[pl.ds(base + c * crows, crows)],
                            idx_v.at[c])
        gcps = [pltpu.async_copy(src_hbm.at[idx_v.at[c]], rows_v.at[c], gsem)
                for c in range(nch)]
        wcps = []
        for c in range(nch):
            gcps[c].wait()
            wcps.append(pltpu.async_copy(
                rows_v.at[c], out_hbm.at[pl.ds(base + c * crows, crows)],
                wsem))
        for cp in wcps:
            cp.wait()

    return gather


_sc_gather_tok = _make_sc_gather(2 * S)

_PPW = 2 * S // _NW


def _sc_dispatch_kernel(src_hbm, slots_hbm, out_hbm, idx_v, rows_v, sem):
    wid = lax.axis_index("s") * 2 + lax.axis_index("c")
    pltpu.sync_copy(slots_hbm.at[wid], idx_v)
    tok0 = lax.rem(wid * _PPW, S)
    pltpu.sync_copy(src_hbm.at[pl.ds(tok0, _PPW)], rows_v)
    pltpu.async_copy(rows_v, out_hbm.at[idx_v], sem).wait()


_sc_dispatch = functools.partial(
    pl.kernel,
    out_type=jax.ShapeDtypeStruct((NPAD, D), jnp.float32),
    mesh=plsc.VectorSubcoreMesh(core_axis_name="c", subcore_axis_name="s"),
    scratch_types=[
        pltpu.VMEM((_PPW,), jnp.int32),
        pltpu.VMEM((_PPW, D), jnp.float32),
        pltpu.SemaphoreType.DMA,
    ],
)(_sc_dispatch_kernel)


def _ffn_kernel(be_ref, nv_ref, xs_ref, w1_ref, b1_ref, w2_ref, b2_ref,
                o_ref):
    b = pl.program_id(0)

    @pl.when(b < nv_ref[0])
    def _():
        xs = xs_ref[...]
        h = (jnp.dot(xs, w1_ref[0], preferred_element_type=jnp.float32)
             + b1_ref[0])
        h = 0.5 * h * (1.0 + lax.erf(h * (2.0 ** -0.5)))
        o_ref[...] = (
            jnp.dot(h, w2_ref[0], preferred_element_type=jnp.float32)
            + b2_ref[0])


def _add3_kernel(x1_ref, g0_ref, g1_ref, gw_ref, y_ref):
    w0 = gw_ref[:, 0:1]
    w1 = gw_ref[:, 1:2]
    y_ref[...] = x1_ref[...] + w0 * g0_ref[0] + w1 * g1_ref[0]


def _routing(i1, i2):
    experts = jnp.concatenate([i1, i2])
    onehot = (experts[:, None] == jnp.arange(E, dtype=jnp.int32)[None, :])
    onehot = onehot.astype(jnp.int32)
    rank = jnp.sum((jnp.cumsum(onehot, axis=0) - onehot) * onehot, axis=1)
    counts = jnp.sum(onehot, axis=0)
    pad_counts = ((counts + BLK - 1) // BLK) * BLK
    cum_pad = jnp.cumsum(pad_counts)
    pad_off = jnp.concatenate(
        [jnp.zeros(1, jnp.int32), cum_pad[:-1].astype(jnp.int32)])
    slot = pad_off[experts] + rank
    nvalid = (cum_pad[-1] // BLK).astype(jnp.int32)
    blk_starts = jnp.arange(NBLK, dtype=jnp.int32) * BLK
    block_expert = jnp.clip(
        jnp.searchsorted(cum_pad, blk_starts, side="right"), 0, E - 1
    ).astype(jnp.int32)
    return block_expert, nvalid, slot


def kernel(x, ln1_w, ln1_b, ln2_w, ln2_b, Wqkv, bqkv, Wo, bo, Wg, W1, b1,
           W2, b2):
    x2 = x.reshape(S, D)
    row2 = lambda a: a.reshape(1, -1)

    qkv = pl.pallas_call(
        _qkv_kernel,
        grid=(NQ,),
        in_specs=[
            pl.BlockSpec((BM, D), lambda i: (i, 0)),
            pl.BlockSpec((1, D), lambda i: (0, 0)),
            pl.BlockSpec((1, D), lambda i: (0, 0)),
            pl.BlockSpec((D, 3 * D), lambda i: (0, 0)),
            pl.BlockSpec((1, 3 * D), lambda i: (0, 0)),
        ],
        out_specs=pl.BlockSpec((BM, 3 * D), lambda i: (i, 0)),
        out_shape=jax.ShapeDtypeStruct((S, 3 * D), jnp.float32),
    )(x2, row2(ln1_w), row2(ln1_b), Wqkv, row2(bqkv))

    qkvr = qkv.reshape(S, 3, H, DH).transpose(1, 2, 0, 3)
    q3, k3, v3 = qkvr[0], qkvr[1], qkvr[2]
    ctxh = pl.pallas_call(
        _attn_kernel,
        grid=(H, NQA),
        in_specs=[
            pl.BlockSpec((1, BMA, DH), lambda h, qi: (h, qi, 0)),
            pl.BlockSpec((1, S, DH), lambda h, qi: (h, 0, 0)),
            pl.BlockSpec((1, S, DH), lambda h, qi: (h, 0, 0)),
        ],
        out_specs=pl.BlockSpec((1, BMA, DH), lambda h, qi: (h, qi, 0)),
        out_shape=jax.ShapeDtypeStruct((H, S, DH), jnp.float32),
    )(q3, k3, v3)
    ctx = ctxh.transpose(1, 0, 2).reshape(S, D)

    Wg_pad = jnp.zeros((D, GP), jnp.float32).at[:, :E].set(Wg)
    outs = pl.pallas_call(
        _post_kernel,
        grid=(NQ,),
        in_specs=[
            pl.BlockSpec((BM, D), lambda i: (i, 0)),
            pl.BlockSpec((BM, D), lambda i: (i, 0)),
            pl.BlockSpec((D, D), lambda i: (0, 0)),
            pl.BlockSpec((1, D), lambda i: (0, 0)),
            pl.BlockSpec((1, D), lambda i: (0, 0)),
            pl.BlockSpec((1, D), lambda i: (0, 0)),
            pl.BlockSpec((D, GP), lambda i: (0, 0)),
        ],
        out_specs=[
            pl.BlockSpec((BM, D), lambda i: (i, 0)),
            pl.BlockSpec((BM, D), lambda i: (i, 0)),
            pl.BlockSpec((BM, 8), lambda i: (i, 0)),
            pl.BlockSpec((BM, 8), lambda i: (i, 0)),
        ],
        out_shape=[
            jax.ShapeDtypeStruct((S, D), jnp.float32),
            jax.ShapeDtypeStruct((S, D), jnp.float32),
            jax.ShapeDtypeStruct((S, 8), jnp.int32),
            jax.ShapeDtypeStruct((S, 8), jnp.float32),
        ],
    )(ctx, x2, Wo, row2(bo), row2(ln2_w), row2(ln2_b), Wg_pad)
    x1, xn2, idxm, gwm = outs

    block_expert, nvalid, slot01 = _routing(idxm[:, 0], idxm[:, 1])
    nv = nvalid.reshape(1)

    xs = _sc_dispatch(xn2, slot01.reshape(_NW, _PPW))

    o = pl.pallas_call(
        _ffn_kernel,
        grid_spec=pltpu.PrefetchScalarGridSpec(
            num_scalar_prefetch=2,
            grid=(NBLK,),
            in_specs=[
                pl.BlockSpec((BLK, D), lambda b, be, nvr: (b, 0)),
                pl.BlockSpec((1, D, F), lambda b, be, nvr: (be[b], 0, 0)),
                pl.BlockSpec((1, 1, F), lambda b, be, nvr: (be[b], 0, 0)),
                pl.BlockSpec((1, F, D), lambda b, be, nvr: (be[b], 0, 0)),
                pl.BlockSpec((1, 1, D), lambda b, be, nvr: (be[b], 0, 0)),
            ],
            out_specs=pl.BlockSpec((BLK, D), lambda b, be, nvr: (b, 0)),
        ),
        out_shape=jax.ShapeDtypeStruct((NPAD, D), jnp.float32),
    )(block_expert, nv, xs, W1, b1.reshape(E, 1, F), W2,
      b2.reshape(E, 1, D))

    g01 = _sc_gather_tok(o, slot01).reshape(2, S, D)

    y = pl.pallas_call(
        _add3_kernel,
        grid=(NQ,),
        in_specs=[
            pl.BlockSpec((BM, D), lambda i: (i, 0)),
            pl.BlockSpec((1, BM, D), lambda i: (0, i, 0)),
            pl.BlockSpec((1, BM, D), lambda i: (1, i, 0)),
            pl.BlockSpec((BM, 8), lambda i: (i, 0)),
        ],
        out_specs=pl.BlockSpec((BM, D), lambda i: (i, 0)),
        out_shape=jax.ShapeDtypeStruct((S, D), jnp.float32),
    )(x1, g01, g01, gwm)

    return y.reshape(1, S, D)


# speedup vs baseline: 1.0163x; 1.0146x over previous
import functools

import jax
import jax.numpy as jnp
from jax import lax
from jax.experimental import pallas as pl
from jax.experimental.pallas import tpu as pltpu
from jax.experimental.pallas import tpu_sc as plsc

S = 2048
D = 768
H = 12
DH = 64
F = 3072
E = 8
BM = 512
BLK = 768
NQ = S // BM
NBLK = (2 * S + E * (BLK - 1) + BLK - 1) // BLK
NPAD = NBLK * BLK
EPS = 1e-5
NEG = -1e9
GP = 128


def _ln(xb, w, b):
    mu = jnp.mean(xb, axis=-1, keepdims=True)
    xc = xb - mu
    var = jnp.mean(xc * xc, axis=-1, keepdims=True)
    return xc * lax.rsqrt(var + EPS) * w + b


def _qkv_kernel(x_ref, lw_ref, lb_ref, wqkv_ref, bqkv_ref, qkv_ref):
    xn = _ln(x_ref[...], lw_ref[...], lb_ref[...])
    qkv_ref[...] = (
        jnp.dot(xn, wqkv_ref[...], preferred_element_type=jnp.float32)
        + bqkv_ref[...]
    )


BMA = 512
NQA = S // BMA


def _attn_kernel(q_ref, k_ref, v_ref, ctx_ref):
    qi = pl.program_id(1)
    q = q_ref[0]
    scale = 1.0 / (DH ** 0.5)

    def make_branch(nb):
        w = nb * BMA

        def br():
            k = k_ref[0, :w, :]
            v = v_ref[0, :w, :]
            s = lax.dot_general(q, k, (((1,), (1,)), ((), ())),
                                preferred_element_type=jnp.float32) * scale
            rows = qi * BMA + lax.broadcasted_iota(jnp.int32, (BMA, w), 0)
            cols = lax.broadcasted_iota(jnp.int32, (BMA, w), 1)
            s = jnp.where(cols <= rows, s, NEG)
            m = jnp.max(s, axis=1, keepdims=True)
            p = jnp.exp(s - m)
            l = jnp.sum(p, axis=1, keepdims=True)
            return jnp.dot(p, v, preferred_element_type=jnp.float32) / l
        return br

    ctx_ref[0] = lax.switch(qi, [make_branch(i + 1) for i in range(NQA)])


def _post_kernel(ctx_ref, x_ref, wo_ref, bo_ref, lw_ref, lb_ref, wg_ref,
                 x1_ref, xn2_ref, idx_ref, gw_ref):
    x1 = (jnp.dot(ctx_ref[...], wo_ref[...],
                  preferred_element_type=jnp.float32)
          + bo_ref[...] + x_ref[...])
    x1_ref[...] = x1
    xn2 = _ln(x1, lw_ref[...], lb_ref[...])
    xn2_ref[...] = xn2
    g = jnp.dot(xn2, wg_ref[...], preferred_element_type=jnp.float32)
    colid = lax.broadcasted_iota(jnp.int32, (BM, GP), 1)
    g = jnp.where(colid < E, g, NEG)
    v1 = jnp.max(g, axis=1, keepdims=True)
    i1 = jnp.min(jnp.where(g == v1, colid, GP), axis=1, keepdims=True)
    gm = jnp.where(colid == i1, NEG, g)
    v2 = jnp.max(gm, axis=1, keepdims=True)
    i2 = jnp.min(jnp.where(gm == v2, colid, GP), axis=1, keepdims=True)
    e2 = jnp.exp(v2 - v1)
    w1 = 1.0 / (1.0 + e2)
    col8 = lax.broadcasted_iota(jnp.int32, (BM, 8), 1)
    idx_ref[...] = jnp.where(col8 == 0, i1, i2)
    gw_ref[...] = jnp.where(col8 == 0, w1, 1.0 - w1)


_NW = 32


def _make_sc_gather(nrows):
    rows_w = nrows // _NW
    nch = -(-rows_w // 40)
    crows = rows_w // nch
    mesh = plsc.VectorSubcoreMesh(core_axis_name="c", subcore_axis_name="s")

    @functools.partial(
        pl.kernel,
        out_type=jax.ShapeDtypeStruct((nrows, D), jnp.float32),
        mesh=mesh,
        scratch_types=[
            pltpu.VMEM((nch, crows), jnp.int32),
            pltpu.VMEM((nch, crows, D), jnp.float32),
            pltpu.SemaphoreType.DMA,
            pltpu.SemaphoreType.DMA,
        ],
    )
    def gather(src_hbm, ids_hbm, out_hbm, idx_v, rows_v, gsem, wsem):
        wid = lax.axis_index("s") * 2 + lax.axis_index("c")
        base = wid * rows_w
        for c in range(nch):
            pltpu.sync_copy(ids_hbm.at[pl.ds(base + c * crows, crows)],
                            idx_v.at[c])
        gcps = [pltpu.async_copy(src_hbm.at[idx_v.at[c]], rows_v.at[c], gsem)
                for c in range(nch)]
        wcps = []
        for c in range(nch):
            gcps[c].wait()
            wcps.append(pltpu.async_copy(
                rows_v.at[c], out_hbm.at[pl.ds(base + c * crows, crows)],
                wsem))
        for cp in wcps:
            cp.wait()

    return gather


_sc_gather_tok = _make_sc_gather(2 * S)

_PPW = 2 * S // _NW


def _sc_dispatch_kernel(src_hbm, slots_hbm, out_hbm, idx_v, rows_v, sem):
    wid = lax.axis_index("s") * 2 + lax.axis_index("c")
    pltpu.sync_copy(slots_hbm.at[wid], idx_v)
    tok0 = lax.rem(wid * _PPW, S)
    pltpu.sync_copy(src_hbm.at[pl.ds(tok0, _PPW)], rows_v)
    pltpu.async_copy(rows_v, out_hbm.at[idx_v], sem).wait()


_sc_dispatch = functools.partial(
    pl.kernel,
    out_type=jax.ShapeDtypeStruct((NPAD, D), jnp.float32),
    mesh=plsc.VectorSubcoreMesh(core_axis_name="c", subcore_axis_name="s"),
    scratch_types=[
        pltpu.VMEM((_PPW,), jnp.int32),
        pltpu.VMEM((_PPW, D), jnp.float32),
        pltpu.SemaphoreType.DMA,
    ],
)(_sc_dispatch_kernel)


def _ffn_kernel(be_ref, nv_ref, xs_ref, w1_ref, b1_ref, w2_ref, b2_ref,
                o_ref):
    b = pl.program_id(0)

    @pl.when(b < nv_ref[0])
    def _():
        xs = xs_ref[...]
        h = (jnp.dot(xs, w1_ref[0], preferred_element_type=jnp.float32)
             + b1_ref[0])
        h = 0.5 * h * (1.0 + lax.erf(h * (2.0 ** -0.5)))
        o_ref[...] = (
            jnp.dot(h, w2_ref[0], preferred_element_type=jnp.float32)
            + b2_ref[0])


def _add3_kernel(x1_ref, g0_ref, g1_ref, gw_ref, y_ref):
    w0 = gw_ref[:, 0:1]
    w1 = gw_ref[:, 1:2]
    y_ref[...] = x1_ref[...] + w0 * g0_ref[0] + w1 * g1_ref[0]


def _routing(i1, i2):
    experts = jnp.concatenate([i1, i2])
    onehot = (experts[:, None] == jnp.arange(E, dtype=jnp.int32)[None, :])
    onehot = onehot.astype(jnp.int32)
    rank = jnp.sum((jnp.cumsum(onehot, axis=0) - onehot) * onehot, axis=1)
    counts = jnp.sum(onehot, axis=0)
    pad_counts = ((counts + BLK - 1) // BLK) * BLK
    cum_pad = jnp.cumsum(pad_counts)
    pad_off = jnp.concatenate(
        [jnp.zeros(1, jnp.int32), cum_pad[:-1].astype(jnp.int32)])
    slot = jnp.sum(onehot * pad_off[None, :], axis=1) + rank
    nvalid = (cum_pad[-1] // BLK).astype(jnp.int32)
    blk_starts = jnp.arange(NBLK, dtype=jnp.int32) * BLK
    block_expert = jnp.clip(
        jnp.searchsorted(cum_pad, blk_starts, side="right"), 0, E - 1
    ).astype(jnp.int32)
    return block_expert, nvalid, slot


def kernel(x, ln1_w, ln1_b, ln2_w, ln2_b, Wqkv, bqkv, Wo, bo, Wg, W1, b1,
           W2, b2):
    x2 = x.reshape(S, D)
    row2 = lambda a: a.reshape(1, -1)

    qkv = pl.pallas_call(
        _qkv_kernel,
        grid=(NQ,),
        in_specs=[
            pl.BlockSpec((BM, D), lambda i: (i, 0)),
            pl.BlockSpec((1, D), lambda i: (0, 0)),
            pl.BlockSpec((1, D), lambda i: (0, 0)),
            pl.BlockSpec((D, 3 * D), lambda i: (0, 0)),
            pl.BlockSpec((1, 3 * D), lambda i: (0, 0)),
        ],
        out_specs=pl.BlockSpec((BM, 3 * D), lambda i: (i, 0)),
        out_shape=jax.ShapeDtypeStruct((S, 3 * D), jnp.float32),
    )(x2, row2(ln1_w), row2(ln1_b), Wqkv, row2(bqkv))

    qkvr = qkv.reshape(S, 3, H, DH).transpose(1, 2, 0, 3)
    q3, k3, v3 = qkvr[0], qkvr[1], qkvr[2]
    ctxh = pl.pallas_call(
        _attn_kernel,
        grid=(H, NQA),
        in_specs=[
            pl.BlockSpec((1, BMA, DH), lambda h, qi: (h, qi, 0)),
            pl.BlockSpec((1, S, DH), lambda h, qi: (h, 0, 0)),
            pl.BlockSpec((1, S, DH), lambda h, qi: (h, 0, 0)),
        ],
        out_specs=pl.BlockSpec((1, BMA, DH), lambda h, qi: (h, qi, 0)),
        out_shape=jax.ShapeDtypeStruct((H, S, DH), jnp.float32),
    )(q3, k3, v3)
    ctx = ctxh.transpose(1, 0, 2).reshape(S, D)

    Wg_pad = jnp.zeros((D, GP), jnp.float32).at[:, :E].set(Wg)
    outs = pl.pallas_call(
        _post_kernel,
        grid=(NQ,),
        in_specs=[
            pl.BlockSpec((BM, D), lambda i: (i, 0)),
            pl.BlockSpec((BM, D), lambda i: (i, 0)),
            pl.BlockSpec((D, D), lambda i: (0, 0)),
            pl.BlockSpec((1, D), lambda i: (0, 0)),
            pl.BlockSpec((1, D), lambda i: (0, 0)),
            pl.BlockSpec((1, D), lambda i: (0, 0)),
            pl.BlockSpec((D, GP), lambda i: (0, 0)),
        ],
        out_specs=[
            pl.BlockSpec((BM, D), lambda i: (i, 0)),
            pl.BlockSpec((BM, D), lambda i: (i, 0)),
            pl.BlockSpec((BM, 8), lambda i: (i, 0)),
            pl.BlockSpec((BM, 8), lambda i: (i, 0)),
        ],
        out_shape=[
            jax.ShapeDtypeStruct((S, D), jnp.float32),
            jax.ShapeDtypeStruct((S, D), jnp.float32),
            jax.ShapeDtypeStruct((S, 8), jnp.int32),
            jax.ShapeDtypeStruct((S, 8), jnp.float32),
        ],
    )(ctx, x2, Wo, row2(bo), row2(ln2_w), row2(ln2_b), Wg_pad)
    x1, xn2, idxm, gwm = outs

    block_expert, nvalid, slot01 = _routing(idxm[:, 0], idxm[:, 1])
    nv = nvalid.reshape(1)

    xs = _sc_dispatch(xn2, slot01.reshape(_NW, _PPW))

    o = pl.pallas_call(
        _ffn_kernel,
        grid_spec=pltpu.PrefetchScalarGridSpec(
            num_scalar_prefetch=2,
            grid=(NBLK,),
            in_specs=[
                pl.BlockSpec((BLK, D), lambda b, be, nvr: (b, 0)),
                pl.BlockSpec((1, D, F), lambda b, be, nvr: (be[b], 0, 0)),
                pl.BlockSpec((1, 1, F), lambda b, be, nvr: (be[b], 0, 0)),
                pl.BlockSpec((1, F, D), lambda b, be, nvr: (be[b], 0, 0)),
                pl.BlockSpec((1, 1, D), lambda b, be, nvr: (be[b], 0, 0)),
            ],
            out_specs=pl.BlockSpec((BLK, D), lambda b, be, nvr: (b, 0)),
        ),
        out_shape=jax.ShapeDtypeStruct((NPAD, D), jnp.float32),
    )(block_expert, nv, xs, W1, b1.reshape(E, 1, F), W2,
      b2.reshape(E, 1, D))

    g01 = _sc_gather_tok(o, slot01).reshape(2, S, D)

    y = pl.pallas_call(
        _add3_kernel,
        grid=(NQ,),
        in_specs=[
            pl.BlockSpec((BM, D), lambda i: (i, 0)),
            pl.BlockSpec((1, BM, D), lambda i: (0, i, 0)),
            pl.BlockSpec((1, BM, D), lambda i: (1, i, 0)),
            pl.BlockSpec((BM, 8), lambda i: (i, 0)),
        ],
        out_specs=pl.BlockSpec((BM, D), lambda i: (i, 0)),
        out_shape=jax.ShapeDtypeStruct((S, D), jnp.float32),
    )(x1, g01, g01, gwm)

    return y.reshape(1, S, D)
